# Initial kernel scaffold; baseline (speedup 1.0000x reference)
import jax
import jax.numpy as jnp
from jax.experimental import pallas as pl


def kernel(x, dist_params):
    raise NotImplementedError("write your pallas kernel here")


# speedup vs baseline: 1.0995x; 1.0995x over previous
import functools

import jax
import jax.numpy as jnp
from jax import lax
from jax.experimental import pallas as pl
from jax.experimental.pallas import tpu as pltpu
from jax.experimental.pallas import tpu_sc as plsc

N_CLS = 256
L = 16
NC, NS = 2, 16
NW = NC * NS

_LN2 = 0.6931471805599453
_SQRT2 = 1.4142135623730951


def _tree(op, xs):
    xs = list(xs)
    while len(xs) > 1:
        nxt = [op(xs[i], xs[i + 1]) for i in range(0, len(xs) - 1, 2)]
        if len(xs) % 2:
            nxt.append(xs[-1])
        xs = nxt
    return xs[0]


def _vlog(s):
    i = lax.bitcast_convert_type(s, jnp.int32)
    e = lax.shift_right_arithmetic(i, 23) - 127
    m = lax.bitcast_convert_type((i & 0x007FFFFF) | 0x3F800000, jnp.float32)
    big = m > _SQRT2
    m = jnp.where(big, 0.5 * m, m)
    ef = (e + jnp.where(big, 1, 0)).astype(jnp.float32)
    t = (m - 1.0) / (m + 1.0)
    t2 = t * t
    p = 2.0 + t2 * (2.0 / 3.0 + t2 * (2.0 / 5.0 + t2 * (2.0 / 7.0 + t2 * (2.0 / 9.0))))
    return ef * _LN2 + t * p


@functools.lru_cache(maxsize=None)
def _build(total_rows):
    assert total_rows % NW == 0
    R = total_rows // NW
    C = 128 if R % 256 == 0 else R // 2
    NCH = R // C
    assert NCH % 2 == 0 and C % L == 0

    mesh = plsc.VectorSubcoreMesh(
        core_axis_name="c", subcore_axis_name="s",
        num_cores=NC, num_subcores=NS)

    @functools.partial(
        pl.kernel,
        out_type=jax.ShapeDtypeStruct((total_rows,), jnp.float32),
        mesh=mesh,
        compiler_params=pltpu.CompilerParams(needs_layout_passes=False),
        scratch_types=[
            pltpu.VMEM((C, N_CLS), jnp.float32),
            pltpu.VMEM((C, N_CLS), jnp.float32),
            pltpu.VMEM((R,), jnp.int32),
            pltpu.VMEM((R,), jnp.float32),
            pltpu.SemaphoreType.DMA,
            pltpu.SemaphoreType.DMA,
        ],
    )
    def sc_kernel(lp_hbm, x_hbm, out_hbm, buf0, buf1, xbuf, obuf, sem0, sem1):
        wid = lax.axis_index("s") * NC + lax.axis_index("c")
        base = wid * R

        pltpu.sync_copy(x_hbm.at[pl.ds(base, R)], xbuf)

        def dma_start(ci, buf, sem):
            pltpu.async_copy(lp_hbm.at[pl.ds(base + ci * C, C)], buf, sem)

        def dma_wait(buf, sem):
            pltpu.make_async_copy(lp_hbm.at[pl.ds(0, C)], buf, sem).wait()

        lane = lax.iota(jnp.int32, L)

        def process(ci, buf):
            def gbody(g, carry):
                row0 = g * L
                acc_m = jnp.zeros((L,), jnp.float32)
                acc_s = jnp.zeros((L,), jnp.float32)
                for t in range(L):
                    vals = [buf[row0 + t, pl.ds(j * L, L)] for j in range(N_CLS // L)]
                    mx = jnp.max(_tree(jnp.maximum, vals))
                    ssum = jnp.sum(_tree(jnp.add, [jnp.exp(v - mx) for v in vals]))
                    sel = lane == t
                    acc_m = jnp.where(sel, mx, acc_m)
                    acc_s = jnp.where(sel, ssum, acc_s)
                lse = acc_m + _vlog(acc_s)
                rows = row0 + lane
                xi = xbuf[pl.ds(ci * C + row0, L)]
                picked = plsc.load_gather(buf, [rows, xi])
                obuf[pl.ds(ci * C + row0, L)] = picked - lse
                return carry

            lax.fori_loop(0, C // L, gbody, 0)

        dma_start(0, buf0, sem0)

        def cbody(k, carry):
            i = 2 * k

            @pl.when(i + 1 < NCH)
            def _():
                dma_start(i + 1, buf1, sem1)

            dma_wait(buf0, sem0)
            process(i, buf0)

            @pl.when(i + 2 < NCH)
            def _():
                dma_start(i + 2, buf0, sem0)

            dma_wait(buf1, sem1)
            process(i + 1, buf1)
            return carry

        lax.fori_loop(0, NCH // 2, cbody, 0)

        pltpu.sync_copy(obuf, out_hbm.at[pl.ds(base, R)])

    return sc_kernel


def kernel(x, dist_params):
    b, t, ch = x.shape
    total = b * t * ch
    lp = dist_params.reshape(total, N_CLS)
    xf = x.reshape(total)
    out = _build(total)(lp, xf)
    return out.reshape(b, t, ch)
